# baseline (speedup 1.0000x reference)
import functools

import jax
import jax.numpy as jnp
from jax import lax
from jax.experimental import pallas as pl
from jax.experimental.pallas import tpu as pltpu
from jax.experimental.pallas import tpu_sc as plsc


@functools.lru_cache(maxsize=None)
def _make_pool_kernel(B: int, L: int, EMB: int):
    NC, NS = 2, 16
    NW = NC * NS
    assert B % NW == 0
    bpw = B // NW
    S = 4
    NBUF = 4
    assert bpw % S == 0
    C = S * L
    G = -(-C // 128)
    CP = G * 128
    NSTEPS = bpw // S
    assert NSTEPS % NBUF == 0
    NJ = NSTEPS // NBUF
    HALF = EMB // 2
    assert EMB == 2 * 16

    mesh = plsc.VectorSubcoreMesh(core_axis_name="c", subcore_axis_name="s")

    scratch = ([pltpu.VMEM((CP,), jnp.int32) for _ in range(NBUF)]
               + [pltpu.VMEM((CP, EMB), jnp.float32) for _ in range(NBUF)]
               + [pltpu.VMEM((S, EMB), jnp.float32)]
               + [pltpu.SemaphoreType.DMA for _ in range(NBUF)])

    @functools.partial(
        pl.kernel,
        out_type=jax.ShapeDtypeStruct((B, EMB), jnp.float32),
        mesh=mesh,
        compiler_params=pltpu.CompilerParams(use_tc_tiling_on_sc=False),
        scratch_types=scratch,
    )
    def pool(emb_hbm, xf_hbm, out_hbm, *sc):
        idxs = sc[:NBUF]
        rows = sc[NBUF:2 * NBUF]
        stage = sc[2 * NBUF]
        sems = sc[2 * NBUF + 1:]
        wid = lax.axis_index("s") * NC + lax.axis_index("c")
        ibase = wid * (bpw * L)
        obase = wid * bpw

        zpad = jnp.zeros((16,), jnp.int32)
        for b in range(NBUF):
            for t in range(C, CP, 16):
                idxs[b][pl.ds(t, 16)] = zpad

        def fire(step, b):
            off = ibase + step * C
            pltpu.sync_copy(xf_hbm.at[pl.ds(off, C)],
                            idxs[b].at[pl.ds(0, C)])
            for g in range(G):
                pltpu.async_copy(
                    emb_hbm.at[idxs[b].at[pl.ds(g * 128, 128)]],
                    rows[b].at[pl.ds(g * 128, 128)],
                    sems[b])

        def wait_all(b):
            pltpu.make_async_copy(
                emb_hbm.at[pl.ds(0, CP)], rows[b], sems[b]).wait()

        inv_l = jnp.float32(1.0 / L)

        UNROLL = 8
        assert L % UNROLL == 0
        NITER = L // UNROLL

        def reduce_step(step, rowsb):
            def body(i, acc):
                accl = list(acc)
                base = i * UNROLL
                for s in range(S):
                    r0 = s * L + base
                    lo = accl[2 * s]
                    hi = accl[2 * s + 1]
                    for u in range(UNROLL):
                        lo = lo + rowsb[r0 + u, pl.ds(0, HALF)]
                        hi = hi + rowsb[r0 + u, pl.ds(HALF, HALF)]
                    accl[2 * s] = lo
                    accl[2 * s + 1] = hi
                return tuple(accl)

            z = jnp.zeros((HALF,), jnp.float32)
            acc = lax.fori_loop(0, NITER, body, (z,) * (2 * S))
            for s in range(S):
                stage[s, pl.ds(0, HALF)] = acc[2 * s] * inv_l
                stage[s, pl.ds(HALF, HALF)] = acc[2 * s + 1] * inv_l
            pltpu.sync_copy(stage, out_hbm.at[pl.ds(obase + step * S, S)])

        for b in range(NBUF):
            fire(b, b)

        def loop_body(j, carry):
            s0 = j * NBUF
            for b in range(NBUF):
                wait_all(b)
                reduce_step(s0 + b, rows[b])

                @pl.when(j < NJ - 1)
                def _():
                    fire(s0 + b + NBUF, b)
            return carry

        lax.fori_loop(0, NJ, loop_body, 0)

    return pool


@functools.lru_cache(maxsize=None)
def _make_mlp_kernel(B: int, EMB: int, HID: int, NCLS: int):
    BB = 1024
    assert B % BB == 0

    def body(p_ref, w1_ref, b1_ref, w2_ref, b2_ref, o_ref):
        h = jnp.dot(p_ref[...], w1_ref[...],
                    preferred_element_type=jnp.float32) + b1_ref[...]
        h = jnp.maximum(h, 0.0)
        o_ref[...] = jnp.dot(h, w2_ref[...],
                             preferred_element_type=jnp.float32) + b2_ref[...]

    return pl.pallas_call(
        body,
        grid=(B // BB,),
        in_specs=[
            pl.BlockSpec((BB, EMB), lambda i: (i, 0)),
            pl.BlockSpec((EMB, HID), lambda i: (0, 0)),
            pl.BlockSpec((1, HID), lambda i: (0, 0)),
            pl.BlockSpec((HID, NCLS), lambda i: (0, 0)),
            pl.BlockSpec((1, NCLS), lambda i: (0, 0)),
        ],
        out_specs=pl.BlockSpec((BB, NCLS), lambda i: (i, 0)),
        out_shape=jax.ShapeDtypeStruct((B, NCLS), jnp.float32),
    )


def kernel(x, emb, W1, b1, W2, b2):
    B, L = x.shape
    EMB = emb.shape[1]
    HID = W1.shape[1]
    NCLS = W2.shape[1]
    pool = _make_pool_kernel(B, L, EMB)
    pooled = pool(emb, x.reshape(B * L))
    mlp = _make_mlp_kernel(B, EMB, HID, NCLS)
    return mlp(pooled, W1, b1.reshape(1, HID), W2, b2.reshape(1, NCLS))


# speedup vs baseline: 2.2242x; 2.2242x over previous
import functools

import jax
import jax.numpy as jnp
from jax import lax
from jax.experimental import pallas as pl
from jax.experimental.pallas import tpu as pltpu
from jax.experimental.pallas import tpu_sc as plsc


@functools.lru_cache(maxsize=None)
def _make_pool_kernel(B: int, L: int, EMB: int):
    NC, NS = 2, 16
    NW = NC * NS
    assert B % NW == 0
    bpw = B // NW
    S = 8
    NBUF = 2
    assert bpw % S == 0
    C = S * L
    G = -(-C // 128)
    CP = G * 128
    NSTEPS = bpw // S
    assert NSTEPS % NBUF == 0
    NJ = NSTEPS // NBUF
    HALF = EMB // 2
    assert EMB == 2 * 16

    mesh = plsc.VectorSubcoreMesh(core_axis_name="c", subcore_axis_name="s")

    scratch = ([pltpu.VMEM((CP,), jnp.int32) for _ in range(NBUF)]
               + [pltpu.VMEM((CP, EMB), jnp.float32) for _ in range(NBUF)]
               + [pltpu.VMEM((S, EMB), jnp.float32)]
               + [pltpu.SemaphoreType.DMA for _ in range(NBUF)])

    @functools.partial(
        pl.kernel,
        out_type=jax.ShapeDtypeStruct((B, EMB), jnp.float32),
        mesh=mesh,
        compiler_params=pltpu.CompilerParams(use_tc_tiling_on_sc=False),
        scratch_types=scratch,
    )
    def pool(emb_hbm, x_hbm, out_hbm, *sc):
        idxs = sc[:NBUF]
        rows = sc[NBUF:2 * NBUF]
        stage = sc[2 * NBUF]
        sems = sc[2 * NBUF + 1:]
        wid = lax.axis_index("s") * NC + lax.axis_index("c")
        obase = wid * bpw

        zpad = jnp.zeros((16,), jnp.int32)
        for b in range(NBUF):
            for t in range(C, CP, 16):
                idxs[b][pl.ds(t, 16)] = zpad

        def fire(step, b):
            row0 = obase + step * S
            for s in range(S):
                pltpu.sync_copy(x_hbm.at[row0 + s],
                                idxs[b].at[pl.ds(s * L, L)])
            for g in range(G):
                pltpu.async_copy(
                    emb_hbm.at[idxs[b].at[pl.ds(g * 128, 128)]],
                    rows[b].at[pl.ds(g * 128, 128)],
                    sems[b])

        def wait_all(b):
            pltpu.make_async_copy(
                emb_hbm.at[pl.ds(0, CP)], rows[b], sems[b]).wait()

        inv_l = jnp.float32(1.0 / L)

        UNROLL = 8
        assert L % UNROLL == 0
        NITER = L // UNROLL

        def reduce_step(step, rowsb):
            def body(i, acc):
                accl = list(acc)
                base = i * UNROLL
                for s in range(S):
                    r0 = s * L + base
                    lo = accl[2 * s]
                    hi = accl[2 * s + 1]
                    for u in range(UNROLL):
                        lo = lo + rowsb[r0 + u, pl.ds(0, HALF)]
                        hi = hi + rowsb[r0 + u, pl.ds(HALF, HALF)]
                    accl[2 * s] = lo
                    accl[2 * s + 1] = hi
                return tuple(accl)

            z = jnp.zeros((HALF,), jnp.float32)
            acc = lax.fori_loop(0, NITER, body, (z,) * (2 * S))
            for s in range(S):
                stage[s, pl.ds(0, HALF)] = acc[2 * s] * inv_l
                stage[s, pl.ds(HALF, HALF)] = acc[2 * s + 1] * inv_l
            pltpu.sync_copy(stage, out_hbm.at[pl.ds(obase + step * S, S)])

        for b in range(NBUF):
            fire(b, b)

        def loop_body(j, carry):
            s0 = j * NBUF
            for b in range(NBUF):
                wait_all(b)
                reduce_step(s0 + b, rows[b])

                @pl.when(j < NJ - 1)
                def _():
                    fire(s0 + b + NBUF, b)
            return carry

        lax.fori_loop(0, NJ, loop_body, 0)

    return pool


@functools.lru_cache(maxsize=None)
def _make_mlp_kernel(B: int, EMB: int, HID: int, NCLS: int):
    BB = 1024
    assert B % BB == 0

    def body(p_ref, w1_ref, b1_ref, w2_ref, b2_ref, o_ref):
        h = jnp.dot(p_ref[...], w1_ref[...],
                    preferred_element_type=jnp.float32) + b1_ref[...]
        h = jnp.maximum(h, 0.0)
        o_ref[...] = jnp.dot(h, w2_ref[...],
                             preferred_element_type=jnp.float32) + b2_ref[...]

    return pl.pallas_call(
        body,
        grid=(B // BB,),
        in_specs=[
            pl.BlockSpec((BB, EMB), lambda i: (i, 0)),
            pl.BlockSpec((EMB, HID), lambda i: (0, 0)),
            pl.BlockSpec((1, HID), lambda i: (0, 0)),
            pl.BlockSpec((HID, NCLS), lambda i: (0, 0)),
            pl.BlockSpec((1, NCLS), lambda i: (0, 0)),
        ],
        out_specs=pl.BlockSpec((BB, NCLS), lambda i: (i, 0)),
        out_shape=jax.ShapeDtypeStruct((B, NCLS), jnp.float32),
    )


def kernel(x, emb, W1, b1, W2, b2):
    B, L = x.shape
    EMB = emb.shape[1]
    HID = W1.shape[1]
    NCLS = W2.shape[1]
    pool = _make_pool_kernel(B, L, EMB)
    pooled = pool(emb, x)
    mlp = _make_mlp_kernel(B, EMB, HID, NCLS)
    return mlp(pooled, W1, b1.reshape(1, HID), W2, b2.reshape(1, NCLS))


# speedup vs baseline: 2.2341x; 1.0044x over previous
import functools

import jax
import jax.numpy as jnp
from jax import lax
from jax.experimental import pallas as pl
from jax.experimental.pallas import tpu as pltpu
from jax.experimental.pallas import tpu_sc as plsc


@functools.lru_cache(maxsize=None)
def _make_pool_kernel(B: int, L: int, EMB: int):
    NC, NS = 2, 16
    NW = NC * NS
    assert B % NW == 0
    bpw = B // NW
    S = 8
    NBUF = 2
    assert bpw % S == 0
    C = S * L
    G = -(-C // 128)
    CP = G * 128
    NSTEPS = bpw // S
    assert NSTEPS % NBUF == 0
    NJ = NSTEPS // NBUF
    HALF = EMB // 2
    assert EMB == 2 * 16

    mesh = plsc.VectorSubcoreMesh(core_axis_name="c", subcore_axis_name="s")

    scratch = ([pltpu.VMEM((CP,), jnp.int32) for _ in range(NBUF)]
               + [pltpu.VMEM((CP, EMB), jnp.float32) for _ in range(NBUF)]
               + [pltpu.VMEM((S, EMB), jnp.float32)]
               + [pltpu.SemaphoreType.DMA for _ in range(NBUF)])

    @functools.partial(
        pl.kernel,
        out_type=jax.ShapeDtypeStruct((B, EMB), jnp.float32),
        mesh=mesh,
        compiler_params=pltpu.CompilerParams(use_tc_tiling_on_sc=False),
        scratch_types=scratch,
    )
    def pool(emb_hbm, xf_hbm, out_hbm, *sc):
        idxs = sc[:NBUF]
        rows = sc[NBUF:2 * NBUF]
        stage = sc[2 * NBUF]
        sems = sc[2 * NBUF + 1:]
        wid = lax.axis_index("s") * NC + lax.axis_index("c")
        obase = wid * bpw

        zpad = jnp.zeros((16,), jnp.int32)
        for b in range(NBUF):
            for t in range(C, CP, 16):
                idxs[b][pl.ds(t, 16)] = zpad

        def fire(step, b):
            off = (obase + step * S) * L
            pltpu.sync_copy(xf_hbm.at[pl.ds(off, C)],
                            idxs[b].at[pl.ds(0, C)])
            for g in range(G):
                pltpu.async_copy(
                    emb_hbm.at[idxs[b].at[pl.ds(g * 128, 128)]],
                    rows[b].at[pl.ds(g * 128, 128)],
                    sems[b])

        def wait_all(b):
            pltpu.make_async_copy(
                emb_hbm.at[pl.ds(0, CP)], rows[b], sems[b]).wait()

        inv_l = jnp.float32(1.0 / L)

        UNROLL = 8
        assert L % UNROLL == 0
        NITER = L // UNROLL

        def reduce_step(step, rowsb):
            def body(i, acc):
                accl = list(acc)
                base = i * UNROLL
                for s in range(S):
                    r0 = s * L + base
                    lo = accl[2 * s]
                    hi = accl[2 * s + 1]
                    for u in range(UNROLL):
                        lo = lo + rowsb[r0 + u, pl.ds(0, HALF)]
                        hi = hi + rowsb[r0 + u, pl.ds(HALF, HALF)]
                    accl[2 * s] = lo
                    accl[2 * s + 1] = hi
                return tuple(accl)

            z = jnp.zeros((HALF,), jnp.float32)
            acc = lax.fori_loop(0, NITER, body, (z,) * (2 * S))
            for s in range(S):
                stage[s, pl.ds(0, HALF)] = acc[2 * s] * inv_l
                stage[s, pl.ds(HALF, HALF)] = acc[2 * s + 1] * inv_l
            pltpu.sync_copy(stage, out_hbm.at[pl.ds(obase + step * S, S)])

        for b in range(NBUF):
            fire(b, b)

        def loop_body(j, carry):
            s0 = j * NBUF
            for b in range(NBUF):
                wait_all(b)
                reduce_step(s0 + b, rows[b])

                @pl.when(j < NJ - 1)
                def _():
                    fire(s0 + b + NBUF, b)
            return carry

        lax.fori_loop(0, NJ, loop_body, 0)

    return pool


@functools.lru_cache(maxsize=None)
def _make_mlp_kernel(B: int, EMB: int, HID: int, NCLS: int):
    BB = 1024
    assert B % BB == 0

    def body(p_ref, w1_ref, b1_ref, w2_ref, b2_ref, o_ref):
        h = jnp.dot(p_ref[...], w1_ref[...],
                    preferred_element_type=jnp.float32) + b1_ref[...]
        h = jnp.maximum(h, 0.0)
        o_ref[...] = jnp.dot(h, w2_ref[...],
                             preferred_element_type=jnp.float32) + b2_ref[...]

    return pl.pallas_call(
        body,
        grid=(B // BB,),
        in_specs=[
            pl.BlockSpec((BB, EMB), lambda i: (i, 0)),
            pl.BlockSpec((EMB, HID), lambda i: (0, 0)),
            pl.BlockSpec((1, HID), lambda i: (0, 0)),
            pl.BlockSpec((HID, NCLS), lambda i: (0, 0)),
            pl.BlockSpec((1, NCLS), lambda i: (0, 0)),
        ],
        out_specs=pl.BlockSpec((BB, NCLS), lambda i: (i, 0)),
        out_shape=jax.ShapeDtypeStruct((B, NCLS), jnp.float32),
    )


def kernel(x, emb, W1, b1, W2, b2):
    B, L = x.shape
    EMB = emb.shape[1]
    HID = W1.shape[1]
    NCLS = W2.shape[1]
    pool = _make_pool_kernel(B, L, EMB)
    pooled = pool(emb, x.reshape(B * L))
    mlp = _make_mlp_kernel(B, EMB, HID, NCLS)
    return mlp(pooled, W1, b1.reshape(1, HID), W2, b2.reshape(1, NCLS))
